# baseline (speedup 1.0000x reference)
import jax
import jax.numpy as jnp
from jax.experimental import pallas as pl
from jax.experimental.pallas import tpu as pltpu

_BM0 = 200
_BM1 = 400
_NBUF = 3
_NBUFB = 2
_NSTAGE = 2
_NCACHE = 4


def _contract_t(t, w):
    return jax.lax.dot_general(
        t, w, (((1,), (1,)), ((), ())), preferred_element_type=jnp.float32
    )


@jax.jit
def kernel(x, adj, W0, W1, Wss):
    n, d = x.shape
    ss = Wss.shape[0]
    f8 = jnp.float8_e4m3fn
    np0 = n // _BM0
    cache_rows = _NCACHE * _BM0
    np1h = (n - cache_rows) // _BM1
    np1 = np1h + cache_rows // _BM1

    def fused_kernel(adj_hbm, x_ref, w0_ref, w1_ref, wss_ref,
                     out_ref, xs_ref, adj8_hbm,
                     bufa, stage, bufb, cache, hf, h8, sema, semw, semb):
        i = pl.program_id(0)

        def _fetch_a(blk, slot):
            return pltpu.make_async_copy(
                adj_hbm.at[pl.ds(blk * _BM0, _BM0), :], bufa.at[slot],
                sema.at[slot])

        def _write8(blk, slot):
            return pltpu.make_async_copy(
                stage.at[slot], adj8_hbm.at[pl.ds(blk * _BM0, _BM0), :],
                semw.at[slot])

        def _fetch_b(blk, slot):
            return pltpu.make_async_copy(
                adj8_hbm.at[pl.ds(blk * _BM1, _BM1), :], bufb.at[slot],
                semb.at[slot])

        @pl.when(i == 0)
        def _prologue():
            for s in range(min(_NBUF, np0)):
                _fetch_a(s, s).start()

        @pl.when((i > 0) & (i + _NBUF - 1 < np0))
        def _lookahead_a():
            blk = i + _NBUF - 1
            _fetch_a(blk, blk % _NBUF).start()

        @pl.when(i < np0)
        def _phase0():
            slot = i % _NBUF
            _fetch_a(i, slot).wait()
            adj_b = bufa[slot].astype(jnp.bfloat16)
            adj8 = adj_b.astype(f8)
            wslot = i % _NSTAGE

            @pl.when(i >= _NSTAGE)
            def _reclaim():
                _write8(i - _NSTAGE, wslot).wait()

            stage[wslot] = adj8
            _write8(i, wslot).start()

            @pl.when(i >= np0 - _NCACHE)
            def _to_cache():
                cache[i - (np0 - _NCACHE)] = adj8

            t = jnp.dot(adj_b, x_ref[...], preferred_element_type=jnp.float32)
            h = jnp.maximum(_contract_t(t, w0_ref[...]), 0.0)
            hf[pl.ds(i * _BM0, _BM0), :] = h

        @pl.when(i >= np0 - 1)
        def _lookahead_b():
            blk = i - np0 + 1
            ok = (blk >= 0) & (blk < np1h)

            @pl.when(ok)
            def _():
                _fetch_b(blk, blk % _NBUFB).start()

        @pl.when(i == np0)
        def _transition():
            for s in range(_NSTAGE):
                blk = np0 - _NSTAGE + s
                _write8(blk, blk % _NSTAGE).wait()
            h8[...] = hf[...].astype(f8)

        @pl.when((i >= np0) & (i < np0 + np1h))
        def _phase1_hbm():
            j = i - np0
            slot = j % _NBUFB
            _fetch_b(j, slot).wait()
            t2 = jnp.dot(bufb[slot], h8[...],
                         preferred_element_type=jnp.float32)
            out_ref[...] = _contract_t(t2, w1_ref[...])
            xs_ref[...] = _contract_t(t2, wss_ref[...])

        @pl.when(i >= np0 + np1h)
        def _phase1_cached():
            jc = i - np0 - np1h
            for s in range(_BM1 // _BM0):
                t2 = jnp.dot(cache[jc * (_BM1 // _BM0) + s], h8[...],
                             preferred_element_type=jnp.float32)
                out_ref[pl.ds(s * _BM0, _BM0), :] = _contract_t(t2, w1_ref[...])
                xs_ref[pl.ds(s * _BM0, _BM0), :] = _contract_t(t2, wss_ref[...])

    full_spec = pl.BlockSpec((n, d), lambda i: (0, 0))
    w_spec = pl.BlockSpec((d, d), lambda i: (0, 0))

    out, xs, _ = pl.pallas_call(
        fused_kernel,
        grid=(np0 + np1,),
        in_specs=[pl.BlockSpec(memory_space=pl.ANY), full_spec, w_spec,
                  w_spec, pl.BlockSpec((ss, d), lambda i: (0, 0))],
        out_specs=[
            pl.BlockSpec((_BM1, d), lambda i: (jnp.maximum(i - np0, 0), 0)),
            pl.BlockSpec((_BM1, ss), lambda i: (jnp.maximum(i - np0, 0), 0)),
            pl.BlockSpec(memory_space=pl.ANY),
        ],
        out_shape=[
            jax.ShapeDtypeStruct((n, d), jnp.float32),
            jax.ShapeDtypeStruct((n, ss), jnp.float32),
            jax.ShapeDtypeStruct((n, n), f8),
        ],
        scratch_shapes=[
            pltpu.VMEM((_NBUF, _BM0, n), jnp.float32),
            pltpu.VMEM((_NSTAGE, _BM0, n), f8),
            pltpu.VMEM((_NBUFB, _BM1, n), f8),
            pltpu.VMEM((_NCACHE, _BM0, n), f8),
            pltpu.VMEM((n, d), jnp.float32),
            pltpu.VMEM((n, d), f8),
            pltpu.SemaphoreType.DMA((_NBUF,)),
            pltpu.SemaphoreType.DMA((_NSTAGE,)),
            pltpu.SemaphoreType.DMA((_NBUFB,)),
        ],
        compiler_params=pltpu.CompilerParams(
            dimension_semantics=("arbitrary",)
        ),
    )(adj, x.astype(jnp.bfloat16), W0, W1, Wss)

    return (out, xs)


# speedup vs baseline: 1.0384x; 1.0384x over previous
import jax
import jax.numpy as jnp
from jax.experimental import pallas as pl
from jax.experimental.pallas import tpu as pltpu

_BM1 = 200
_BM2 = 1000
_NBUF = 3


def _contract_t(t, w):
    return jax.lax.dot_general(
        t, w, (((1,), (1,)), ((), ())), preferred_element_type=jnp.float32
    )


def _pass1_kernel(adj_hbm, x_ref, w0_ref, h_ref, adj8_ref, buf, sem):
    i = pl.program_id(0)
    nsteps = pl.num_programs(0)

    def _fetch(blk, slot):
        return pltpu.make_async_copy(
            adj_hbm.at[pl.ds(blk * _BM1, _BM1), :], buf.at[slot], sem.at[slot]
        )

    @pl.when(i == 0)
    def _prologue():
        for s in range(_NBUF):
            _fetch(s, s).start()

    @pl.when((i > 0) & (i + _NBUF - 1 < nsteps))
    def _next():
        blk = i + _NBUF - 1
        _fetch(blk, blk % _NBUF).start()

    slot = i % _NBUF
    _fetch(i, slot).wait()
    adj_b = buf[slot].astype(jnp.bfloat16)
    adj8_ref[...] = adj_b.astype(jnp.float8_e4m3fn)
    t = jnp.dot(adj_b, x_ref[...], preferred_element_type=jnp.float32)
    h = jnp.maximum(_contract_t(t, w0_ref[...]), 0.0)
    h_ref[...] = h.astype(jnp.float8_e4m3fn)


def _pass2_kernel(adj8_ref, h_ref, w1_ref, wss_ref, out_ref, xs_ref):
    t2 = jnp.dot(adj8_ref[...], h_ref[...], preferred_element_type=jnp.float32)
    out_ref[...] = _contract_t(t2, w1_ref[...])
    xs_ref[...] = _contract_t(t2, wss_ref[...])


@jax.jit
def kernel(x, adj, W0, W1, Wss):
    n, d = x.shape
    ss = Wss.shape[0]
    full_spec = pl.BlockSpec((n, d), lambda i: (0, 0))
    w_spec = pl.BlockSpec((d, d), lambda i: (0, 0))
    params = pltpu.CompilerParams(dimension_semantics=("arbitrary",))

    h8, adj8 = pl.pallas_call(
        _pass1_kernel,
        grid=(n // _BM1,),
        in_specs=[pl.BlockSpec(memory_space=pl.ANY), full_spec, w_spec],
        out_specs=[
            pl.BlockSpec((_BM1, d), lambda i: (i, 0)),
            pl.BlockSpec((_BM1, n), lambda i: (i, 0)),
        ],
        out_shape=[
            jax.ShapeDtypeStruct((n, d), jnp.float8_e4m3fn),
            jax.ShapeDtypeStruct((n, n), jnp.float8_e4m3fn),
        ],
        scratch_shapes=[
            pltpu.VMEM((_NBUF, _BM1, n), jnp.float32),
            pltpu.SemaphoreType.DMA((_NBUF,)),
        ],
        compiler_params=params,
    )(adj, x.astype(jnp.bfloat16), W0)

    out, xs = pl.pallas_call(
        _pass2_kernel,
        grid=(n // _BM2,),
        in_specs=[pl.BlockSpec((_BM2, n), lambda i: (i, 0)), full_spec, w_spec,
                  pl.BlockSpec((ss, d), lambda i: (0, 0))],
        out_specs=[pl.BlockSpec((_BM2, d), lambda i: (i, 0)),
                   pl.BlockSpec((_BM2, ss), lambda i: (i, 0))],
        out_shape=[
            jax.ShapeDtypeStruct((n, d), jnp.float32),
            jax.ShapeDtypeStruct((n, ss), jnp.float32),
        ],
        compiler_params=params,
    )(adj8, h8, W1, Wss)

    return (out, xs)
